# baseline (speedup 1.0000x reference)
import jax
import jax.numpy as jnp
from jax.experimental import pallas as pl
from jax.experimental.pallas import tpu as pltpu

_EPS = 1e-05
_BN = 1024
_K = 8


def _bn_kernel(x_hbm, m_hbm, g_ref, b_ref, mu_ref, var_ref, o_hbm,
               xbuf, mbuf, sin, sout):
    n = x_hbm.shape[0]
    ngroups = n // (_BN * _K)

    inv = jax.lax.rsqrt(var_ref[...] + _EPS)
    scale = g_ref[...] * inv
    bias = b_ref[...] - mu_ref[...] * scale

    def body(g, carry):
        base = g * _K * _BN

        def fire_in(k, cc):
            row = base + k * _BN
            pltpu.make_async_copy(
                x_hbm.at[pl.ds(row, _BN), :], xbuf.at[k], sin.at[k]).start()
            pltpu.make_async_copy(
                m_hbm.at[pl.ds(row, _BN), :], mbuf.at[k], sin.at[k]).start()
            return cc

        jax.lax.fori_loop(0, _K, fire_in, 0)

        def compute(k, cc):
            pltpu.make_async_copy(
                x_hbm.at[pl.ds(base, _BN), :], xbuf.at[k], sin.at[k]).wait()
            pltpu.make_async_copy(
                m_hbm.at[pl.ds(base, _BN), :], mbuf.at[k], sin.at[k]).wait()
            x = xbuf[k]
            m = mbuf[k]
            normed = x * scale + bias
            xbuf[k] = x + m * (normed - x)
            return cc

        jax.lax.fori_loop(0, _K, compute, 0)

        def fire_out(k, cc):
            row = base + k * _BN
            pltpu.make_async_copy(
                xbuf.at[k], o_hbm.at[pl.ds(row, _BN), :], sout.at[k]).start()
            return cc

        jax.lax.fori_loop(0, _K, fire_out, 0)

        def drain_out(k, cc):
            pltpu.make_async_copy(
                xbuf.at[k], o_hbm.at[pl.ds(base, _BN), :], sout.at[k]).wait()
            return cc

        jax.lax.fori_loop(0, _K, drain_out, 0)
        return carry

    jax.lax.fori_loop(0, ngroups, body, 0)


def kernel(x_flat_nc, mask_flat, gamma, beta, moving_mean, moving_var):
    n, c = x_flat_nc.shape
    m2d = mask_flat.astype(jnp.float32)[:, None]
    return pl.pallas_call(
        _bn_kernel,
        in_specs=[
            pl.BlockSpec(memory_space=pl.ANY),
            pl.BlockSpec(memory_space=pl.ANY),
            pl.BlockSpec(memory_space=pltpu.VMEM),
            pl.BlockSpec(memory_space=pltpu.VMEM),
            pl.BlockSpec(memory_space=pltpu.VMEM),
            pl.BlockSpec(memory_space=pltpu.VMEM),
        ],
        out_specs=pl.BlockSpec(memory_space=pl.ANY),
        out_shape=jax.ShapeDtypeStruct((n, c), x_flat_nc.dtype),
        scratch_shapes=[
            pltpu.VMEM((_K, _BN, c), jnp.float32),
            pltpu.VMEM((_K, _BN, 1), jnp.float32),
            pltpu.SemaphoreType.DMA((_K,)),
            pltpu.SemaphoreType.DMA((_K,)),
        ],
    )(x_flat_nc, m2d, gamma[None, :], beta[None, :],
      moving_mean[None, :], moving_var[None, :])


# speedup vs baseline: 1.0974x; 1.0974x over previous
import jax
import jax.numpy as jnp
from jax.experimental import pallas as pl
from jax.experimental.pallas import tpu as pltpu

_EPS = 1e-05
_BN = 512
_NBUF = 6


def _bn_kernel(x_hbm, m_hbm, g_ref, b_ref, mu_ref, var_ref, o_hbm,
               xbuf, mbuf, obuf, in_sem, m_sem, out_sem):
    n = x_hbm.shape[0]
    g = n // _BN

    inv = jax.lax.rsqrt(var_ref[...] + _EPS)
    scale = g_ref[...] * inv
    bias = b_ref[...] - mu_ref[...] * scale

    def in_copy(i, slot):
        return (
            pltpu.make_async_copy(
                x_hbm.at[pl.ds(i * _BN, _BN), :], xbuf.at[slot], in_sem.at[slot]),
            pltpu.make_async_copy(
                m_hbm.at[pl.ds(i * _BN, _BN), :], mbuf.at[slot], m_sem.at[slot]),
        )

    def out_copy(slot):
        return pltpu.make_async_copy(
            obuf.at[slot], o_hbm.at[pl.ds(0, _BN), :], out_sem.at[slot])

    for i in range(_NBUF - 1):
        cx, cm = in_copy(i, i % _NBUF)
        cx.start()
        cm.start()

    def body(i, _):
        slot = jax.lax.rem(i, _NBUF)
        cx, cm = in_copy(i, slot)
        cx.wait()
        cm.wait()

        @pl.when(i >= _NBUF)
        def _():
            out_copy(slot).wait()

        x = xbuf[slot]
        m = mbuf[slot]
        normed = x * scale + bias
        obuf[slot] = x + m * (normed - x)

        pltpu.make_async_copy(
            obuf.at[slot], o_hbm.at[pl.ds(i * _BN, _BN), :], out_sem.at[slot]
        ).start()

        @pl.when(i + _NBUF - 1 < g)
        def _():
            nslot = jax.lax.rem(i + _NBUF - 1, _NBUF)
            nx, nm = in_copy(i + _NBUF - 1, nslot)
            nx.start()
            nm.start()

        return 0

    jax.lax.fori_loop(0, g, body, 0)

    for i in range(max(g - _NBUF, 0), g):
        out_copy(i % _NBUF).wait()


def kernel(x_flat_nc, mask_flat, gamma, beta, moving_mean, moving_var):
    n, c = x_flat_nc.shape
    m2d = mask_flat.astype(jnp.float32)[:, None]
    return pl.pallas_call(
        _bn_kernel,
        in_specs=[
            pl.BlockSpec(memory_space=pl.ANY),
            pl.BlockSpec(memory_space=pl.ANY),
            pl.BlockSpec(memory_space=pltpu.VMEM),
            pl.BlockSpec(memory_space=pltpu.VMEM),
            pl.BlockSpec(memory_space=pltpu.VMEM),
            pl.BlockSpec(memory_space=pltpu.VMEM),
        ],
        out_specs=pl.BlockSpec(memory_space=pl.ANY),
        out_shape=jax.ShapeDtypeStruct((n, c), x_flat_nc.dtype),
        scratch_shapes=[
            pltpu.VMEM((_NBUF, _BN, c), jnp.float32),
            pltpu.VMEM((_NBUF, _BN, 1), jnp.float32),
            pltpu.VMEM((_NBUF, _BN, c), jnp.float32),
            pltpu.SemaphoreType.DMA((_NBUF,)),
            pltpu.SemaphoreType.DMA((_NBUF,)),
            pltpu.SemaphoreType.DMA((_NBUF,)),
        ],
    )(x_flat_nc, m2d, gamma[None, :], beta[None, :],
      moving_mean[None, :], moving_var[None, :])


# speedup vs baseline: 1.1046x; 1.0066x over previous
import jax
import jax.numpy as jnp
from jax.experimental import pallas as pl
from jax.experimental.pallas import tpu as pltpu

_EPS = 1e-05
_BLOCK_N = 4096


def _bn_kernel(x_ref, m_ref, g_ref, b_ref, mu_ref, var_ref, o_ref):
    inv = jax.lax.rsqrt(var_ref[...] + _EPS)
    scale = g_ref[...] * inv
    bias = b_ref[...] - mu_ref[...] * scale
    x = x_ref[...]
    m = m_ref[...]
    normed = x * scale + bias
    o_ref[...] = x + m * (normed - x)


def kernel(x_flat_nc, mask_flat, gamma, beta, moving_mean, moving_var):
    n, c = x_flat_nc.shape
    bn = _BLOCK_N
    m2d = mask_flat.astype(jnp.float32)[:, None]
    g2d = gamma[None, :]
    b2d = beta[None, :]
    mu2d = moving_mean[None, :]
    var2d = moving_var[None, :]
    grid = (n // bn,)
    return pl.pallas_call(
        _bn_kernel,
        grid=grid,
        in_specs=[
            pl.BlockSpec((bn, c), lambda i: (i, 0)),
            pl.BlockSpec((bn, 1), lambda i: (i, 0)),
            pl.BlockSpec((1, c), lambda i: (0, 0)),
            pl.BlockSpec((1, c), lambda i: (0, 0)),
            pl.BlockSpec((1, c), lambda i: (0, 0)),
            pl.BlockSpec((1, c), lambda i: (0, 0)),
        ],
        out_specs=pl.BlockSpec((bn, c), lambda i: (i, 0)),
        out_shape=jax.ShapeDtypeStruct((n, c), x_flat_nc.dtype),
        compiler_params=pltpu.CompilerParams(
            dimension_semantics=("parallel",),
        ),
    )(x_flat_nc, m2d, g2d, b2d, mu2d, var2d)
